# Initial kernel scaffold; baseline (speedup 1.0000x reference)
import jax
import jax.numpy as jnp
from jax.experimental import pallas as pl


def kernel(x, edge_index, W1, b1, g1, be1, W2, b2, g2, be2, W3, b3):
    raise NotImplementedError("write your pallas kernel here")


# speedup vs baseline: 14.3194x; 14.3194x over previous
import functools

import jax
import jax.numpy as jnp
from jax import lax
from jax.experimental import pallas as pl
from jax.experimental.pallas import tpu as pltpu
from jax.experimental.pallas import tpu_sc as plsc

N = 10000
NPAD = 10016
CHUNK = 128
NCHUNKS = 2560
EPAD = NCHUNKS * CHUNK
EPS = 1e-5

_mesh = lambda: plsc.VectorSubcoreMesh(core_axis_name="c", subcore_axis_name="s")


def _split_rows(mk_src, mk_dst, sid, total_rows):
    main = (total_rows // 16) & ~7
    off = sid * main
    pltpu.sync_copy(mk_src(off, main), mk_dst(off, main))
    rem = total_rows - main * 16
    if rem:
        @pl.when(sid == 15)
        def _():
            pltpu.sync_copy(mk_src(main * 16, rem), mk_dst(main * 16, rem))


def _deg_call(dst2d, zeros16, ones16):
    npc = NCHUNKS // 32

    @functools.partial(
        pl.kernel,
        out_type=jax.ShapeDtypeStruct((2, NPAD, 16), jnp.float32),
        mesh=_mesh(),
        scratch_types=[
            pltpu.VMEM((npc, CHUNK), jnp.int32),
            pltpu.VMEM((CHUNK, 16), jnp.float32),
            pltpu.VMEM_SHARED((NPAD, 16), jnp.float32),
        ],
    )
    def deg_kernel(dst_hbm, z_hbm, ones_hbm, out_hbm, dst_v, ones_v, deg_sh):
        cid = lax.axis_index("c")
        sid = lax.axis_index("s")
        pltpu.sync_copy(dst_hbm.at[pl.ds(cid * (NCHUNKS // 2) + sid * npc, npc)],
                        dst_v)
        pltpu.sync_copy(ones_hbm, ones_v)
        _split_rows(lambda o, s: z_hbm.at[pl.ds(o, s)],
                    lambda o, s: deg_sh.at[pl.ds(o, s)], sid, NPAD)
        plsc.subcore_barrier()

        def body(j, carry):
            pltpu.sync_copy(ones_v, deg_sh.at[dst_v.at[j]], add=True)
            return carry

        lax.fori_loop(0, npc, body, 0)
        plsc.subcore_barrier()
        _split_rows(lambda o, s: deg_sh.at[pl.ds(o, s)],
                    lambda o, s: out_hbm.at[cid, pl.ds(o, s)], sid, NPAD)

    return deg_kernel(dst2d, zeros16, ones16)


def _agg_call(src2d, dst2d, hs0, hs1, C):
    npc = NCHUNKS // 16
    GRP = 16
    ngrp = npc // GRP

    @functools.partial(
        pl.kernel,
        out_type=(jax.ShapeDtypeStruct((N, C), jnp.float32),
                  jax.ShapeDtypeStruct((N, C), jnp.float32)),
        mesh=_mesh(),
        scratch_types=[
            pltpu.VMEM((GRP, CHUNK), jnp.int32),
            pltpu.VMEM((GRP, CHUNK), jnp.int32),
            pltpu.VMEM((CHUNK, C), jnp.float32),
            pltpu.VMEM_SHARED((NPAD, C), jnp.float32),
            pltpu.SemaphoreType.DMA,
        ],
    )
    def agg_kernel(src_hbm, dst_hbm, hs0_hbm, hs1_hbm, out0_hbm, out1_hbm,
                   src_v, dst_v, rows_v, agg_sh, sem):
        cid = lax.axis_index("c")
        sid = lax.axis_index("s")

        def run(hs_hbm, out_hbm):
            base = sid * npc
            _split_rows(lambda o, s: hs_hbm.at[pl.ds(o, s)],
                        lambda o, s: agg_sh.at[pl.ds(o, s)], sid, N)
            plsc.subcore_barrier()

            def group(g, carry):
                pltpu.sync_copy(src_hbm.at[pl.ds(base + g * GRP, GRP)], src_v)
                pltpu.sync_copy(dst_hbm.at[pl.ds(base + g * GRP, GRP)], dst_v)

                def body(j, carry2):
                    pltpu.async_copy(hs_hbm.at[src_v.at[j]], rows_v, sem).wait()
                    pltpu.sync_copy(rows_v, agg_sh.at[dst_v.at[j]], add=True)
                    return carry2

                return lax.fori_loop(0, GRP, body, carry)

            lax.fori_loop(0, ngrp, group, 0)
            plsc.subcore_barrier()
            _split_rows(lambda o, s: agg_sh.at[pl.ds(o, s)],
                        lambda o, s: out_hbm.at[pl.ds(o, s)], sid, N)

        @pl.when(cid == 0)
        def _():
            run(hs0_hbm, out0_hbm)

        @pl.when(cid == 1)
        def _():
            run(hs1_hbm, out1_hbm)

    return agg_kernel(src2d, dst2d, hs0, hs1)


def _agg_edge_call(src2d, dst2d, hs):
    C = hs.shape[1]
    npc = NCHUNKS // 32
    GRP = 16
    ngrp = npc // GRP

    @functools.partial(
        pl.kernel,
        out_type=jax.ShapeDtypeStruct((2, N, C), jnp.float32),
        mesh=_mesh(),
        scratch_types=[
            pltpu.VMEM((GRP, CHUNK), jnp.int32),
            pltpu.VMEM((GRP, CHUNK), jnp.int32),
            pltpu.VMEM((CHUNK, C), jnp.float32),
            pltpu.VMEM_SHARED((NPAD, C), jnp.float32),
            pltpu.SemaphoreType.DMA,
        ],
    )
    def agg_kernel(src_hbm, dst_hbm, hs_hbm, out_hbm,
                   src_v, dst_v, rows_v, agg_sh, sem):
        cid = lax.axis_index("c")
        sid = lax.axis_index("s")
        base = cid * (NCHUNKS // 2) + sid * npc
        _split_rows(lambda o, s: hs_hbm.at[pl.ds(o, s)],
                    lambda o, s: agg_sh.at[pl.ds(o, s)], sid, N)
        plsc.subcore_barrier()

        def group(g, carry):
            pltpu.sync_copy(src_hbm.at[pl.ds(base + g * GRP, GRP)], src_v)
            pltpu.sync_copy(dst_hbm.at[pl.ds(base + g * GRP, GRP)], dst_v)

            def body(j, carry2):
                pltpu.async_copy(hs_hbm.at[src_v.at[j]], rows_v, sem).wait()
                pltpu.sync_copy(rows_v, agg_sh.at[dst_v.at[j]], add=True)
                return carry2

            return lax.fori_loop(0, GRP, body, carry)

        lax.fori_loop(0, ngrp, group, 0)
        plsc.subcore_barrier()
        _split_rows(lambda o, s: agg_sh.at[pl.ds(o, s)],
                    lambda o, s: out_hbm.at[cid, pl.ds(o, s)], sid, N)

    return agg_kernel(src2d, dst2d, hs)


def _tc_first(x, W1, degs):
    D = W1.shape[1]
    C = D // 2

    def body(x_ref, w_ref, deg_ref, hs0_ref, hs1_ref, dinv_ref):
        deg = deg_ref[0, 0:N, 0:1] + deg_ref[1, 0:N, 0:1] + 1.0
        dinv = lax.rsqrt(deg)
        h = jnp.dot(x_ref[...], w_ref[...], preferred_element_type=jnp.float32)
        hs = h * dinv
        hs0_ref[...] = hs[:, :C]
        hs1_ref[...] = hs[:, C:]
        dinv_ref[...] = dinv

    return pl.pallas_call(
        body,
        out_shape=(jax.ShapeDtypeStruct((N, C), jnp.float32),
                   jax.ShapeDtypeStruct((N, C), jnp.float32),
                   jax.ShapeDtypeStruct((N, 1), jnp.float32)),
    )(x, W1, degs)


def _tc_mid(a0, a1, dinv, b, g, be, W, split):
    Dn = W.shape[1]
    Cn = Dn // 2

    def body(a0_ref, a1_ref, dinv_ref, b_ref, g_ref, be_ref, w_ref, *outs):
        dinv = dinv_ref[...]
        z = jnp.concatenate([a0_ref[...], a1_ref[...]], axis=1) * dinv + b_ref[...]
        mean = jnp.mean(z, axis=0, keepdims=True)
        ctr = z - mean
        var = jnp.mean(ctr * ctr, axis=0, keepdims=True)
        y = jnp.maximum(ctr * lax.rsqrt(var + EPS) * g_ref[...] + be_ref[...],
                        0.0)
        h = jnp.dot(y, w_ref[...], preferred_element_type=jnp.float32)
        hs = h * dinv
        if split:
            outs[0][...] = hs[:, :Cn]
            outs[1][...] = hs[:, Cn:]
        else:
            outs[0][...] = hs

    if split:
        out_shape = (jax.ShapeDtypeStruct((N, Cn), jnp.float32),
                     jax.ShapeDtypeStruct((N, Cn), jnp.float32))
    else:
        out_shape = jax.ShapeDtypeStruct((N, Dn), jnp.float32)
    return pl.pallas_call(body, out_shape=out_shape)(a0, a1, dinv, b, g, be, W)


def _tc_last(parts, hs, dinv, b):
    D = hs.shape[1]

    def body(p_ref, hs_ref, dinv_ref, b_ref, out_ref):
        agg = p_ref[0] + p_ref[1] - hs_ref[...]
        out_ref[...] = agg * dinv_ref[...] + b_ref[...]

    return pl.pallas_call(
        body,
        out_shape=jax.ShapeDtypeStruct((N, D), jnp.float32),
    )(parts, hs, dinv, b)


def kernel(x, edge_index, W1, b1, g1, be1, W2, b2, g2, be2, W3, b3):
    src = edge_index[0].astype(jnp.int32)
    dst = edge_index[1].astype(jnp.int32)
    npad_e = EPAD - src.shape[0]
    fill = jnp.arange(npad_e, dtype=jnp.int32) % 16
    src2d = jnp.concatenate([src, fill]).reshape(NCHUNKS, CHUNK)
    dst2d = jnp.concatenate([dst, N + fill]).reshape(NCHUNKS, CHUNK)

    zeros16 = jnp.zeros((NPAD, 16), jnp.float32)
    ones16 = jnp.ones((CHUNK, 16), jnp.float32)
    degs = _deg_call(dst2d, zeros16, ones16)

    b1r, g1r, be1r = b1.reshape(1, -1), g1.reshape(1, -1), be1.reshape(1, -1)
    b2r, g2r, be2r = b2.reshape(1, -1), g2.reshape(1, -1), be2.reshape(1, -1)
    b3r = b3.reshape(1, -1)

    hs0, hs1, dinv = _tc_first(x, W1, degs)
    a0, a1 = _agg_call(src2d, dst2d, hs0, hs1, hs0.shape[1])
    hs0, hs1 = _tc_mid(a0, a1, dinv, b1r, g1r, be1r, W2, split=True)
    a0, a1 = _agg_call(src2d, dst2d, hs0, hs1, hs0.shape[1])
    hs3 = _tc_mid(a0, a1, dinv, b2r, g2r, be2r, W3, split=False)
    parts = _agg_edge_call(src2d, dst2d, hs3)
    return _tc_last(parts, hs3, dinv, b3r)


# speedup vs baseline: 19.5689x; 1.3666x over previous
import functools

import jax
import jax.numpy as jnp
from jax import lax
from jax.experimental import pallas as pl
from jax.experimental.pallas import tpu as pltpu
from jax.experimental.pallas import tpu_sc as plsc

N = 10000
NPAD = 10016
CHUNK = 128
NCHUNKS = 2560
EPAD = NCHUNKS * CHUNK
EPS = 1e-5

_mesh = lambda: plsc.VectorSubcoreMesh(core_axis_name="c", subcore_axis_name="s")


def _split_rows(mk_src, mk_dst, sid, total_rows):
    main = (total_rows // 16) & ~7
    off = sid * main
    pltpu.sync_copy(mk_src(off, main), mk_dst(off, main))
    rem = total_rows - main * 16
    if rem:
        @pl.when(sid == 15)
        def _():
            pltpu.sync_copy(mk_src(main * 16, rem), mk_dst(main * 16, rem))


def _edge_pipeline(hs_hbm, agg_sh, src_hbm, dst_hbm, src_v, dst_v, rows,
                   sem_ab, base, grp, ngrp):
    rows_a, rows_b = rows
    sem_a, sem_b = sem_ab
    bufs = (rows_a, rows_b)
    sems = (sem_a, sem_b)

    def group(g, carry):
        pltpu.sync_copy(src_hbm.at[pl.ds(base + g * grp, grp)], src_v)
        pltpu.sync_copy(dst_hbm.at[pl.ds(base + g * grp, grp)], dst_v)
        cp = pltpu.async_copy(hs_hbm.at[src_v.at[0]], bufs[0], sems[0])
        for j in range(grp):
            if j + 1 < grp:
                cp_next = pltpu.async_copy(hs_hbm.at[src_v.at[j + 1]],
                                           bufs[(j + 1) & 1], sems[(j + 1) & 1])
            cp.wait()
            pltpu.sync_copy(bufs[j & 1], agg_sh.at[dst_v.at[j]], add=True)
            if j + 1 < grp:
                cp = cp_next
        return carry

    lax.fori_loop(0, ngrp, group, 0)


def _deg_call(dst2d, zeros16, ones16):
    npc = NCHUNKS // 32

    @functools.partial(
        pl.kernel,
        out_type=jax.ShapeDtypeStruct((2, NPAD, 128), jnp.float32),
        mesh=_mesh(),
        scratch_types=[
            pltpu.VMEM((npc, CHUNK), jnp.int32),
            pltpu.VMEM((CHUNK, 128), jnp.float32),
            pltpu.VMEM_SHARED((NPAD, 128), jnp.float32),
        ],
    )
    def deg_kernel(dst_hbm, z_hbm, ones_hbm, out_hbm, dst_v, ones_v, deg_sh):
        cid = lax.axis_index("c")
        sid = lax.axis_index("s")
        pltpu.sync_copy(dst_hbm.at[pl.ds(cid * (NCHUNKS // 2) + sid * npc, npc)],
                        dst_v)
        pltpu.sync_copy(ones_hbm, ones_v)
        _split_rows(lambda o, s: z_hbm.at[pl.ds(o, s)],
                    lambda o, s: deg_sh.at[pl.ds(o, s)], sid, NPAD)
        plsc.subcore_barrier()

        def body(j, carry):
            pltpu.sync_copy(ones_v, deg_sh.at[dst_v.at[j]], add=True)
            return carry

        lax.fori_loop(0, npc, body, 0)
        plsc.subcore_barrier()
        _split_rows(lambda o, s: deg_sh.at[pl.ds(o, s)],
                    lambda o, s: out_hbm.at[cid, pl.ds(o, s)], sid, NPAD)

    return deg_kernel(dst2d, zeros16, ones16)


def _agg_call(src2d, dst2d, hs0, hs1, C):
    npc = NCHUNKS // 16
    GRP = 16
    ngrp = npc // GRP

    @functools.partial(
        pl.kernel,
        out_type=(jax.ShapeDtypeStruct((N, C), jnp.float32),
                  jax.ShapeDtypeStruct((N, C), jnp.float32)),
        mesh=_mesh(),
        scratch_types=[
            pltpu.VMEM((GRP, CHUNK), jnp.int32),
            pltpu.VMEM((GRP, CHUNK), jnp.int32),
            pltpu.VMEM((CHUNK, C), jnp.float32),
            pltpu.VMEM((CHUNK, C), jnp.float32),
            pltpu.VMEM_SHARED((NPAD, C), jnp.float32),
            pltpu.SemaphoreType.DMA,
            pltpu.SemaphoreType.DMA,
        ],
    )
    def agg_kernel(src_hbm, dst_hbm, hs0_hbm, hs1_hbm, out0_hbm, out1_hbm,
                   src_v, dst_v, rows_a, rows_b, agg_sh, sem_a, sem_b):
        cid = lax.axis_index("c")
        sid = lax.axis_index("s")

        def run(hs_hbm, out_hbm):
            base = sid * npc
            _split_rows(lambda o, s: hs_hbm.at[pl.ds(o, s)],
                        lambda o, s: agg_sh.at[pl.ds(o, s)], sid, N)
            plsc.subcore_barrier()
            _edge_pipeline(hs_hbm, agg_sh, src_hbm, dst_hbm, src_v, dst_v,
                           (rows_a, rows_b), (sem_a, sem_b), base, GRP, ngrp)
            plsc.subcore_barrier()
            _split_rows(lambda o, s: agg_sh.at[pl.ds(o, s)],
                        lambda o, s: out_hbm.at[pl.ds(o, s)], sid, N)

        @pl.when(cid == 0)
        def _():
            run(hs0_hbm, out0_hbm)

        @pl.when(cid == 1)
        def _():
            run(hs1_hbm, out1_hbm)

    return agg_kernel(src2d, dst2d, hs0, hs1)


def _agg_edge_call(src2d, dst2d, hs):
    C = hs.shape[1]
    npc = NCHUNKS // 32
    GRP = 16
    ngrp = npc // GRP

    @functools.partial(
        pl.kernel,
        out_type=jax.ShapeDtypeStruct((2, N, C), jnp.float32),
        mesh=_mesh(),
        scratch_types=[
            pltpu.VMEM((GRP, CHUNK), jnp.int32),
            pltpu.VMEM((GRP, CHUNK), jnp.int32),
            pltpu.VMEM((CHUNK, C), jnp.float32),
            pltpu.VMEM((CHUNK, C), jnp.float32),
            pltpu.VMEM_SHARED((NPAD, C), jnp.float32),
            pltpu.SemaphoreType.DMA,
            pltpu.SemaphoreType.DMA,
        ],
    )
    def agg_kernel(src_hbm, dst_hbm, hs_hbm, out_hbm,
                   src_v, dst_v, rows_a, rows_b, agg_sh, sem_a, sem_b):
        cid = lax.axis_index("c")
        sid = lax.axis_index("s")
        base = cid * (NCHUNKS // 2) + sid * npc
        _split_rows(lambda o, s: hs_hbm.at[pl.ds(o, s)],
                    lambda o, s: agg_sh.at[pl.ds(o, s)], sid, N)
        plsc.subcore_barrier()
        _edge_pipeline(hs_hbm, agg_sh, src_hbm, dst_hbm, src_v, dst_v,
                       (rows_a, rows_b), (sem_a, sem_b), base, GRP, ngrp)
        plsc.subcore_barrier()
        _split_rows(lambda o, s: agg_sh.at[pl.ds(o, s)],
                    lambda o, s: out_hbm.at[cid, pl.ds(o, s)], sid, N)

    return agg_kernel(src2d, dst2d, hs)


def _tc_first(x, W1, degs):
    D = W1.shape[1]
    C = D // 2

    def body(x_ref, w_ref, deg_ref, hs0_ref, hs1_ref, dinv_ref):
        deg = deg_ref[0, 0:N, 0:1] + deg_ref[1, 0:N, 0:1] + 1.0
        dinv = lax.rsqrt(deg)
        h = jnp.dot(x_ref[...], w_ref[...], preferred_element_type=jnp.float32)
        hs = h * dinv
        hs0_ref[...] = hs[:, :C]
        hs1_ref[...] = hs[:, C:]
        dinv_ref[...] = dinv

    return pl.pallas_call(
        body,
        out_shape=(jax.ShapeDtypeStruct((N, C), jnp.float32),
                   jax.ShapeDtypeStruct((N, C), jnp.float32),
                   jax.ShapeDtypeStruct((N, 1), jnp.float32)),
    )(x, W1, degs)


def _tc_mid(a0, a1, dinv, b, g, be, W, split):
    Dn = W.shape[1]
    Cn = Dn // 2

    def body(a0_ref, a1_ref, dinv_ref, b_ref, g_ref, be_ref, w_ref, *outs):
        dinv = dinv_ref[...]
        z = jnp.concatenate([a0_ref[...], a1_ref[...]], axis=1) * dinv + b_ref[...]
        mean = jnp.mean(z, axis=0, keepdims=True)
        ctr = z - mean
        var = jnp.mean(ctr * ctr, axis=0, keepdims=True)
        y = jnp.maximum(ctr * lax.rsqrt(var + EPS) * g_ref[...] + be_ref[...],
                        0.0)
        h = jnp.dot(y, w_ref[...], preferred_element_type=jnp.float32)
        hs = h * dinv
        if split:
            outs[0][...] = hs[:, :Cn]
            outs[1][...] = hs[:, Cn:]
        else:
            outs[0][...] = hs

    if split:
        out_shape = (jax.ShapeDtypeStruct((N, Cn), jnp.float32),
                     jax.ShapeDtypeStruct((N, Cn), jnp.float32))
    else:
        out_shape = jax.ShapeDtypeStruct((N, Dn), jnp.float32)
    return pl.pallas_call(body, out_shape=out_shape)(a0, a1, dinv, b, g, be, W)


def _tc_last(parts, hs, dinv, b):
    D = hs.shape[1]

    def body(p_ref, hs_ref, dinv_ref, b_ref, out_ref):
        agg = p_ref[0] + p_ref[1] - hs_ref[...]
        out_ref[...] = agg * dinv_ref[...] + b_ref[...]

    return pl.pallas_call(
        body,
        out_shape=jax.ShapeDtypeStruct((N, D), jnp.float32),
    )(parts, hs, dinv, b)


def kernel(x, edge_index, W1, b1, g1, be1, W2, b2, g2, be2, W3, b3):
    src = edge_index[0].astype(jnp.int32)
    dst = edge_index[1].astype(jnp.int32)
    npad_e = EPAD - src.shape[0]
    fill = jnp.arange(npad_e, dtype=jnp.int32) % 16
    src2d = jnp.concatenate([src, fill]).reshape(NCHUNKS, CHUNK)
    dst2d = jnp.concatenate([dst, N + fill]).reshape(NCHUNKS, CHUNK)

    zeros128 = jnp.zeros((NPAD, 128), jnp.float32)
    ones128 = jnp.ones((CHUNK, 128), jnp.float32)
    degs = _deg_call(dst2d, zeros128, ones128)

    b1r, g1r, be1r = b1.reshape(1, -1), g1.reshape(1, -1), be1.reshape(1, -1)
    b2r, g2r, be2r = b2.reshape(1, -1), g2.reshape(1, -1), be2.reshape(1, -1)
    b3r = b3.reshape(1, -1)

    hs0, hs1, dinv = _tc_first(x, W1, degs)
    a0, a1 = _agg_call(src2d, dst2d, hs0, hs1, hs0.shape[1])
    hs0, hs1 = _tc_mid(a0, a1, dinv, b1r, g1r, be1r, W2, split=True)
    a0, a1 = _agg_call(src2d, dst2d, hs0, hs1, hs0.shape[1])
    hs3 = _tc_mid(a0, a1, dinv, b2r, g2r, be2r, W3, split=False)
    parts = _agg_edge_call(src2d, dst2d, hs3)
    return _tc_last(parts, hs3, dinv, b3r)


# speedup vs baseline: 20.9604x; 1.0711x over previous
import functools

import jax
import jax.numpy as jnp
from jax import lax
from jax.experimental import pallas as pl
from jax.experimental.pallas import tpu as pltpu
from jax.experimental.pallas import tpu_sc as plsc

N = 10000
NPAD = 10016
CHUNK = 128
NCHUNKS = 2560
EPAD = NCHUNKS * CHUNK
EPS = 1e-5

_mesh = lambda: plsc.VectorSubcoreMesh(core_axis_name="c", subcore_axis_name="s")


def _split_rows(mk_src, mk_dst, sid, total_rows):
    main = (total_rows // 16) & ~7
    off = sid * main
    pltpu.sync_copy(mk_src(off, main), mk_dst(off, main))
    rem = total_rows - main * 16
    if rem:
        @pl.when(sid == 15)
        def _():
            pltpu.sync_copy(mk_src(main * 16, rem), mk_dst(main * 16, rem))


def _edge_pipeline(hs_hbm, agg_sh, src_hbm, dst_hbm, idxbufs, rows,
                   sem_ab, sem_i, base, grp, ngrp):
    (bufs, sems) = rows, sem_ab
    (s0, d0), (s1, d1) = idxbufs
    pltpu.sync_copy(src_hbm.at[pl.ds(base, grp)], s0)
    pltpu.sync_copy(dst_hbm.at[pl.ds(base, grp)], d0)
    cp = pltpu.async_copy(hs_hbm.at[s0.at[0]], bufs[0], sems[0])
    for g in range(ngrp):
        cur_s, cur_d = idxbufs[g & 1]
        if g + 1 < ngrp:
            nxt_s, nxt_d = idxbufs[(g + 1) & 1]
            cpi_s = pltpu.async_copy(
                src_hbm.at[pl.ds(base + (g + 1) * grp, grp)], nxt_s, sem_i)
            cpi_d = pltpu.async_copy(
                dst_hbm.at[pl.ds(base + (g + 1) * grp, grp)], nxt_d, sem_i)
        for j in range(grp):
            k = g * grp + j
            if j + 1 < grp:
                cp_next = pltpu.async_copy(hs_hbm.at[cur_s.at[j + 1]],
                                           bufs[(k + 1) & 1],
                                           sems[(k + 1) & 1])
            elif g + 1 < ngrp:
                cpi_s.wait()
                cpi_d.wait()
                cp_next = pltpu.async_copy(hs_hbm.at[nxt_s.at[0]],
                                           bufs[(k + 1) & 1],
                                           sems[(k + 1) & 1])
            else:
                cp_next = None
            cp.wait()
            pltpu.sync_copy(bufs[k & 1], agg_sh.at[cur_d.at[j]], add=True)
            cp = cp_next


def _deg_call(dst2d, zeros16, ones16):
    npc = NCHUNKS // 32

    @functools.partial(
        pl.kernel,
        out_type=jax.ShapeDtypeStruct((2, NPAD, 128), jnp.float32),
        mesh=_mesh(),
        scratch_types=[
            pltpu.VMEM((npc, CHUNK), jnp.int32),
            pltpu.VMEM((CHUNK, 128), jnp.float32),
            pltpu.VMEM_SHARED((NPAD, 128), jnp.float32),
        ],
    )
    def deg_kernel(dst_hbm, z_hbm, ones_hbm, out_hbm, dst_v, ones_v, deg_sh):
        cid = lax.axis_index("c")
        sid = lax.axis_index("s")
        pltpu.sync_copy(dst_hbm.at[pl.ds(cid * (NCHUNKS // 2) + sid * npc, npc)],
                        dst_v)
        pltpu.sync_copy(ones_hbm, ones_v)
        _split_rows(lambda o, s: z_hbm.at[pl.ds(o, s)],
                    lambda o, s: deg_sh.at[pl.ds(o, s)], sid, NPAD)
        plsc.subcore_barrier()

        def body(j, carry):
            pltpu.sync_copy(ones_v, deg_sh.at[dst_v.at[j]], add=True)
            return carry

        lax.fori_loop(0, npc, body, 0)
        plsc.subcore_barrier()
        _split_rows(lambda o, s: deg_sh.at[pl.ds(o, s)],
                    lambda o, s: out_hbm.at[cid, pl.ds(o, s)], sid, NPAD)

    return deg_kernel(dst2d, zeros16, ones16)


def _agg_call(src2d, dst2d, hs0, hs1, C):
    npc = NCHUNKS // 16
    GRP = 16
    ngrp = npc // GRP

    @functools.partial(
        pl.kernel,
        out_type=(jax.ShapeDtypeStruct((N, C), jnp.float32),
                  jax.ShapeDtypeStruct((N, C), jnp.float32)),
        mesh=_mesh(),
        scratch_types=[
            pltpu.VMEM((GRP, CHUNK), jnp.int32),
            pltpu.VMEM((GRP, CHUNK), jnp.int32),
            pltpu.VMEM((GRP, CHUNK), jnp.int32),
            pltpu.VMEM((GRP, CHUNK), jnp.int32),
            pltpu.VMEM((CHUNK, C), jnp.float32),
            pltpu.VMEM((CHUNK, C), jnp.float32),
            pltpu.VMEM_SHARED((NPAD, C), jnp.float32),
            pltpu.SemaphoreType.DMA,
            pltpu.SemaphoreType.DMA,
            pltpu.SemaphoreType.DMA,
        ],
    )
    def agg_kernel(src_hbm, dst_hbm, hs0_hbm, hs1_hbm, out0_hbm, out1_hbm,
                   s0_v, d0_v, s1_v, d1_v, rows_a, rows_b, agg_sh, sem_a, sem_b, sem_i):
        cid = lax.axis_index("c")
        sid = lax.axis_index("s")

        def run(hs_hbm, out_hbm):
            base = sid * npc
            _split_rows(lambda o, s: hs_hbm.at[pl.ds(o, s)],
                        lambda o, s: agg_sh.at[pl.ds(o, s)], sid, N)
            plsc.subcore_barrier()
            _edge_pipeline(hs_hbm, agg_sh, src_hbm, dst_hbm,
                           ((s0_v, d0_v), (s1_v, d1_v)), (rows_a, rows_b),
                           (sem_a, sem_b), sem_i, base, GRP, ngrp)
            plsc.subcore_barrier()
            _split_rows(lambda o, s: agg_sh.at[pl.ds(o, s)],
                        lambda o, s: out_hbm.at[pl.ds(o, s)], sid, N)

        @pl.when(cid == 0)
        def _():
            run(hs0_hbm, out0_hbm)

        @pl.when(cid == 1)
        def _():
            run(hs1_hbm, out1_hbm)

    return agg_kernel(src2d, dst2d, hs0, hs1)


def _agg_edge_call(src2d, dst2d, hs):
    C = hs.shape[1]
    npc = NCHUNKS // 32
    GRP = 16
    ngrp = npc // GRP

    @functools.partial(
        pl.kernel,
        out_type=jax.ShapeDtypeStruct((2, N, C), jnp.float32),
        mesh=_mesh(),
        scratch_types=[
            pltpu.VMEM((GRP, CHUNK), jnp.int32),
            pltpu.VMEM((GRP, CHUNK), jnp.int32),
            pltpu.VMEM((GRP, CHUNK), jnp.int32),
            pltpu.VMEM((GRP, CHUNK), jnp.int32),
            pltpu.VMEM((CHUNK, C), jnp.float32),
            pltpu.VMEM((CHUNK, C), jnp.float32),
            pltpu.VMEM_SHARED((NPAD, C), jnp.float32),
            pltpu.SemaphoreType.DMA,
            pltpu.SemaphoreType.DMA,
            pltpu.SemaphoreType.DMA,
        ],
    )
    def agg_kernel(src_hbm, dst_hbm, hs_hbm, out_hbm,
                   s0_v, d0_v, s1_v, d1_v, rows_a, rows_b, agg_sh, sem_a, sem_b, sem_i):
        cid = lax.axis_index("c")
        sid = lax.axis_index("s")
        base = cid * (NCHUNKS // 2) + sid * npc
        _split_rows(lambda o, s: hs_hbm.at[pl.ds(o, s)],
                    lambda o, s: agg_sh.at[pl.ds(o, s)], sid, N)
        plsc.subcore_barrier()
        _edge_pipeline(hs_hbm, agg_sh, src_hbm, dst_hbm,
                       ((s0_v, d0_v), (s1_v, d1_v)), (rows_a, rows_b),
                       (sem_a, sem_b), sem_i, base, GRP, ngrp)
        plsc.subcore_barrier()
        _split_rows(lambda o, s: agg_sh.at[pl.ds(o, s)],
                    lambda o, s: out_hbm.at[cid, pl.ds(o, s)], sid, N)

    return agg_kernel(src2d, dst2d, hs)


def _tc_first(x, W1, degs):
    D = W1.shape[1]
    C = D // 2

    def body(x_ref, w_ref, deg_ref, hs0_ref, hs1_ref, dinv_ref):
        deg = deg_ref[0, 0:N, 0:1] + deg_ref[1, 0:N, 0:1] + 1.0
        dinv = lax.rsqrt(deg)
        h = jnp.dot(x_ref[...], w_ref[...], preferred_element_type=jnp.float32)
        hs = h * dinv
        hs0_ref[...] = hs[:, :C]
        hs1_ref[...] = hs[:, C:]
        dinv_ref[...] = dinv

    return pl.pallas_call(
        body,
        out_shape=(jax.ShapeDtypeStruct((N, C), jnp.float32),
                   jax.ShapeDtypeStruct((N, C), jnp.float32),
                   jax.ShapeDtypeStruct((N, 1), jnp.float32)),
    )(x, W1, degs)


def _tc_mid(a0, a1, dinv, b, g, be, W, split):
    Dn = W.shape[1]
    Cn = Dn // 2

    def body(a0_ref, a1_ref, dinv_ref, b_ref, g_ref, be_ref, w_ref, *outs):
        dinv = dinv_ref[...]
        z = jnp.concatenate([a0_ref[...], a1_ref[...]], axis=1) * dinv + b_ref[...]
        mean = jnp.mean(z, axis=0, keepdims=True)
        ctr = z - mean
        var = jnp.mean(ctr * ctr, axis=0, keepdims=True)
        y = jnp.maximum(ctr * lax.rsqrt(var + EPS) * g_ref[...] + be_ref[...],
                        0.0)
        h = jnp.dot(y, w_ref[...], preferred_element_type=jnp.float32)
        hs = h * dinv
        if split:
            outs[0][...] = hs[:, :Cn]
            outs[1][...] = hs[:, Cn:]
        else:
            outs[0][...] = hs

    if split:
        out_shape = (jax.ShapeDtypeStruct((N, Cn), jnp.float32),
                     jax.ShapeDtypeStruct((N, Cn), jnp.float32))
    else:
        out_shape = jax.ShapeDtypeStruct((N, Dn), jnp.float32)
    return pl.pallas_call(body, out_shape=out_shape)(a0, a1, dinv, b, g, be, W)


def _tc_last(parts, hs, dinv, b):
    D = hs.shape[1]

    def body(p_ref, hs_ref, dinv_ref, b_ref, out_ref):
        agg = p_ref[0] + p_ref[1] - hs_ref[...]
        out_ref[...] = agg * dinv_ref[...] + b_ref[...]

    return pl.pallas_call(
        body,
        out_shape=jax.ShapeDtypeStruct((N, D), jnp.float32),
    )(parts, hs, dinv, b)


def kernel(x, edge_index, W1, b1, g1, be1, W2, b2, g2, be2, W3, b3):
    src = edge_index[0].astype(jnp.int32)
    dst = edge_index[1].astype(jnp.int32)
    npad_e = EPAD - src.shape[0]
    fill = jnp.arange(npad_e, dtype=jnp.int32) % 16
    src2d = jnp.concatenate([src, fill]).reshape(NCHUNKS, CHUNK)
    dst2d = jnp.concatenate([dst, N + fill]).reshape(NCHUNKS, CHUNK)

    zeros128 = jnp.zeros((NPAD, 128), jnp.float32)
    ones128 = jnp.ones((CHUNK, 128), jnp.float32)
    degs = _deg_call(dst2d, zeros128, ones128)

    b1r, g1r, be1r = b1.reshape(1, -1), g1.reshape(1, -1), be1.reshape(1, -1)
    b2r, g2r, be2r = b2.reshape(1, -1), g2.reshape(1, -1), be2.reshape(1, -1)
    b3r = b3.reshape(1, -1)

    hs0, hs1, dinv = _tc_first(x, W1, degs)
    a0, a1 = _agg_call(src2d, dst2d, hs0, hs1, hs0.shape[1])
    hs0, hs1 = _tc_mid(a0, a1, dinv, b1r, g1r, be1r, W2, split=True)
    a0, a1 = _agg_call(src2d, dst2d, hs0, hs1, hs0.shape[1])
    hs3 = _tc_mid(a0, a1, dinv, b2r, g2r, be2r, W3, split=False)
    parts = _agg_edge_call(src2d, dst2d, hs3)
    return _tc_last(parts, hs3, dinv, b3r)


# speedup vs baseline: 20.9914x; 1.0015x over previous
import functools

import jax
import jax.numpy as jnp
from jax import lax
from jax.experimental import pallas as pl
from jax.experimental.pallas import tpu as pltpu
from jax.experimental.pallas import tpu_sc as plsc

N = 10000
NPAD = 10016
CHUNK = 128
NCHUNKS = 2560
EPAD = NCHUNKS * CHUNK
EPS = 1e-5

_mesh = lambda: plsc.VectorSubcoreMesh(core_axis_name="c", subcore_axis_name="s")


def _split_rows(mk_src, mk_dst, sid, total_rows):
    main = (total_rows // 16) & ~7
    off = sid * main
    pltpu.sync_copy(mk_src(off, main), mk_dst(off, main))
    rem = total_rows - main * 16
    if rem:
        @pl.when(sid == 15)
        def _():
            pltpu.sync_copy(mk_src(main * 16, rem), mk_dst(main * 16, rem))


def _edge_pipeline(hs_hbm, agg_sh, src_hbm, dst_hbm, idxbufs, rows,
                   sem_ab, sem_i, base, grp, ngrp):
    (bufs, sems) = rows, sem_ab
    (s0, d0), (s1, d1) = idxbufs
    pltpu.sync_copy(src_hbm.at[pl.ds(base, grp)], s0)
    pltpu.sync_copy(dst_hbm.at[pl.ds(base, grp)], d0)
    cp = pltpu.async_copy(hs_hbm.at[s0.at[0]], bufs[0], sems[0])
    for g in range(ngrp):
        cur_s, cur_d = idxbufs[g & 1]
        if g + 1 < ngrp:
            nxt_s, nxt_d = idxbufs[(g + 1) & 1]
            cpi_s = pltpu.async_copy(
                src_hbm.at[pl.ds(base + (g + 1) * grp, grp)], nxt_s, sem_i)
            cpi_d = pltpu.async_copy(
                dst_hbm.at[pl.ds(base + (g + 1) * grp, grp)], nxt_d, sem_i)
        for j in range(grp):
            k = g * grp + j
            if j + 1 < grp:
                cp_next = pltpu.async_copy(hs_hbm.at[cur_s.at[j + 1]],
                                           bufs[(k + 1) & 1],
                                           sems[(k + 1) & 1])
            elif g + 1 < ngrp:
                cpi_s.wait()
                cpi_d.wait()
                cp_next = pltpu.async_copy(hs_hbm.at[nxt_s.at[0]],
                                           bufs[(k + 1) & 1],
                                           sems[(k + 1) & 1])
            else:
                cp_next = None
            cp.wait()
            pltpu.sync_copy(bufs[k & 1], agg_sh.at[cur_d.at[j]], add=True)
            cp = cp_next


def _deg_call(dst2d, zeros128, ones128):
    npc = NCHUNKS // 32

    @functools.partial(
        pl.kernel,
        out_type=jax.ShapeDtypeStruct((2, NPAD, 128), jnp.float32),
        mesh=_mesh(),
        scratch_types=[
            pltpu.VMEM((npc, CHUNK), jnp.int32),
            pltpu.VMEM((CHUNK, 128), jnp.float32),
            pltpu.VMEM_SHARED((NPAD, 128), jnp.float32),
            pltpu.SemaphoreType.DMA,
        ],
    )
    def deg_kernel(dst_hbm, z_hbm, ones_hbm, out_hbm, dst_v, ones_v, deg_sh,
                   sem_s):
        cid = lax.axis_index("c")
        sid = lax.axis_index("s")
        pltpu.sync_copy(dst_hbm.at[pl.ds(cid * (NCHUNKS // 2) + sid * npc, npc)],
                        dst_v)
        pltpu.sync_copy(ones_hbm, ones_v)
        _split_rows(lambda o, s: z_hbm.at[pl.ds(o, s)],
                    lambda o, s: deg_sh.at[pl.ds(o, s)], sid, NPAD)
        plsc.subcore_barrier()

        cps = []
        for j in range(npc):
            cps.append(pltpu.async_copy(ones_v, deg_sh.at[dst_v.at[j]],
                                        sem_s, add=True))
            if len(cps) == 8:
                for cp in cps:
                    cp.wait()
                cps = []
        for cp in cps:
            cp.wait()
        plsc.subcore_barrier()
        _split_rows(lambda o, s: deg_sh.at[pl.ds(o, s)],
                    lambda o, s: out_hbm.at[cid, pl.ds(o, s)], sid, NPAD)

    return deg_kernel(dst2d, zeros128, ones128)


def _agg_call(src2d, dst2d, hs0, hs1, C):
    npc = NCHUNKS // 16
    GRP = 16
    ngrp = npc // GRP

    @functools.partial(
        pl.kernel,
        out_type=(jax.ShapeDtypeStruct((N, C), jnp.float32),
                  jax.ShapeDtypeStruct((N, C), jnp.float32)),
        mesh=_mesh(),
        scratch_types=[
            pltpu.VMEM((GRP, CHUNK), jnp.int32),
            pltpu.VMEM((GRP, CHUNK), jnp.int32),
            pltpu.VMEM((GRP, CHUNK), jnp.int32),
            pltpu.VMEM((GRP, CHUNK), jnp.int32),
            pltpu.VMEM((CHUNK, C), jnp.float32),
            pltpu.VMEM((CHUNK, C), jnp.float32),
            pltpu.VMEM_SHARED((NPAD, C), jnp.float32),
            pltpu.SemaphoreType.DMA,
            pltpu.SemaphoreType.DMA,
            pltpu.SemaphoreType.DMA,
        ],
    )
    def agg_kernel(src_hbm, dst_hbm, hs0_hbm, hs1_hbm, out0_hbm, out1_hbm,
                   s0_v, d0_v, s1_v, d1_v, rows_a, rows_b, agg_sh, sem_a, sem_b, sem_i):
        cid = lax.axis_index("c")
        sid = lax.axis_index("s")

        def run(hs_hbm, out_hbm):
            base = sid * npc
            _split_rows(lambda o, s: hs_hbm.at[pl.ds(o, s)],
                        lambda o, s: agg_sh.at[pl.ds(o, s)], sid, N)
            plsc.subcore_barrier()
            _edge_pipeline(hs_hbm, agg_sh, src_hbm, dst_hbm,
                           ((s0_v, d0_v), (s1_v, d1_v)), (rows_a, rows_b),
                           (sem_a, sem_b), sem_i, base, GRP, ngrp)
            plsc.subcore_barrier()
            _split_rows(lambda o, s: agg_sh.at[pl.ds(o, s)],
                        lambda o, s: out_hbm.at[pl.ds(o, s)], sid, N)

        @pl.when(cid == 0)
        def _():
            run(hs0_hbm, out0_hbm)

        @pl.when(cid == 1)
        def _():
            run(hs1_hbm, out1_hbm)

    return agg_kernel(src2d, dst2d, hs0, hs1)


def _agg_edge_call(src2d, dst2d, hs):
    C = hs.shape[1]
    npc = NCHUNKS // 32
    GRP = 16
    ngrp = npc // GRP

    @functools.partial(
        pl.kernel,
        out_type=jax.ShapeDtypeStruct((2, N, C), jnp.float32),
        mesh=_mesh(),
        scratch_types=[
            pltpu.VMEM((GRP, CHUNK), jnp.int32),
            pltpu.VMEM((GRP, CHUNK), jnp.int32),
            pltpu.VMEM((GRP, CHUNK), jnp.int32),
            pltpu.VMEM((GRP, CHUNK), jnp.int32),
            pltpu.VMEM((CHUNK, C), jnp.float32),
            pltpu.VMEM((CHUNK, C), jnp.float32),
            pltpu.VMEM_SHARED((NPAD, C), jnp.float32),
            pltpu.SemaphoreType.DMA,
            pltpu.SemaphoreType.DMA,
            pltpu.SemaphoreType.DMA,
        ],
    )
    def agg_kernel(src_hbm, dst_hbm, hs_hbm, out_hbm,
                   s0_v, d0_v, s1_v, d1_v, rows_a, rows_b, agg_sh, sem_a, sem_b, sem_i):
        cid = lax.axis_index("c")
        sid = lax.axis_index("s")
        base = cid * (NCHUNKS // 2) + sid * npc
        _split_rows(lambda o, s: hs_hbm.at[pl.ds(o, s)],
                    lambda o, s: agg_sh.at[pl.ds(o, s)], sid, N)
        plsc.subcore_barrier()
        _edge_pipeline(hs_hbm, agg_sh, src_hbm, dst_hbm,
                       ((s0_v, d0_v), (s1_v, d1_v)), (rows_a, rows_b),
                       (sem_a, sem_b), sem_i, base, GRP, ngrp)
        plsc.subcore_barrier()
        _split_rows(lambda o, s: agg_sh.at[pl.ds(o, s)],
                    lambda o, s: out_hbm.at[cid, pl.ds(o, s)], sid, N)

    return agg_kernel(src2d, dst2d, hs)


def _tc_first(x, W1, degs):
    D = W1.shape[1]
    C = D // 2

    def body(x_ref, w_ref, deg_ref, hs0_ref, hs1_ref, dinv_ref):
        deg = deg_ref[0, 0:N, 0:1] + deg_ref[1, 0:N, 0:1] + 1.0
        dinv = lax.rsqrt(deg)
        h = jnp.dot(x_ref[...], w_ref[...], preferred_element_type=jnp.float32)
        hs = h * dinv
        hs0_ref[...] = hs[:, :C]
        hs1_ref[...] = hs[:, C:]
        dinv_ref[...] = dinv

    return pl.pallas_call(
        body,
        out_shape=(jax.ShapeDtypeStruct((N, C), jnp.float32),
                   jax.ShapeDtypeStruct((N, C), jnp.float32),
                   jax.ShapeDtypeStruct((N, 1), jnp.float32)),
    )(x, W1, degs)


def _tc_mid(a0, a1, dinv, b, g, be, W, split):
    Dn = W.shape[1]
    Cn = Dn // 2

    def body(a0_ref, a1_ref, dinv_ref, b_ref, g_ref, be_ref, w_ref, *outs):
        dinv = dinv_ref[...]
        z = jnp.concatenate([a0_ref[...], a1_ref[...]], axis=1) * dinv + b_ref[...]
        mean = jnp.mean(z, axis=0, keepdims=True)
        ctr = z - mean
        var = jnp.mean(ctr * ctr, axis=0, keepdims=True)
        y = jnp.maximum(ctr * lax.rsqrt(var + EPS) * g_ref[...] + be_ref[...],
                        0.0)
        h = jnp.dot(y, w_ref[...], preferred_element_type=jnp.float32)
        hs = h * dinv
        if split:
            outs[0][...] = hs[:, :Cn]
            outs[1][...] = hs[:, Cn:]
        else:
            outs[0][...] = hs

    if split:
        out_shape = (jax.ShapeDtypeStruct((N, Cn), jnp.float32),
                     jax.ShapeDtypeStruct((N, Cn), jnp.float32))
    else:
        out_shape = jax.ShapeDtypeStruct((N, Dn), jnp.float32)
    return pl.pallas_call(body, out_shape=out_shape)(a0, a1, dinv, b, g, be, W)


def _tc_last(parts, hs, dinv, b):
    D = hs.shape[1]

    def body(p_ref, hs_ref, dinv_ref, b_ref, out_ref):
        agg = p_ref[0] + p_ref[1] - hs_ref[...]
        out_ref[...] = agg * dinv_ref[...] + b_ref[...]

    return pl.pallas_call(
        body,
        out_shape=jax.ShapeDtypeStruct((N, D), jnp.float32),
    )(parts, hs, dinv, b)


def kernel(x, edge_index, W1, b1, g1, be1, W2, b2, g2, be2, W3, b3):
    src = edge_index[0].astype(jnp.int32)
    dst = edge_index[1].astype(jnp.int32)
    npad_e = EPAD - src.shape[0]
    fill = jnp.arange(npad_e, dtype=jnp.int32) % 16
    src2d = jnp.concatenate([src, fill]).reshape(NCHUNKS, CHUNK)
    dst2d = jnp.concatenate([dst, N + fill]).reshape(NCHUNKS, CHUNK)

    zeros128 = jnp.zeros((NPAD, 128), jnp.float32)
    ones128 = jnp.ones((CHUNK, 128), jnp.float32)
    degs = _deg_call(dst2d, zeros128, ones128)

    b1r, g1r, be1r = b1.reshape(1, -1), g1.reshape(1, -1), be1.reshape(1, -1)
    b2r, g2r, be2r = b2.reshape(1, -1), g2.reshape(1, -1), be2.reshape(1, -1)
    b3r = b3.reshape(1, -1)

    hs0, hs1, dinv = _tc_first(x, W1, degs)
    a0, a1 = _agg_call(src2d, dst2d, hs0, hs1, hs0.shape[1])
    hs0, hs1 = _tc_mid(a0, a1, dinv, b1r, g1r, be1r, W2, split=True)
    a0, a1 = _agg_call(src2d, dst2d, hs0, hs1, hs0.shape[1])
    hs3 = _tc_mid(a0, a1, dinv, b2r, g2r, be2r, W3, split=False)
    parts = _agg_edge_call(src2d, dst2d, hs3)
    return _tc_last(parts, hs3, dinv, b3r)
